# Initial kernel scaffold; baseline (speedup 1.0000x reference)
import jax
import jax.numpy as jnp
from jax.experimental import pallas as pl


def kernel(q, k, v, associations, ln_gamma, ln_beta, Wq, bq, Wk, bk, Wo, bo):
    raise NotImplementedError("write your pallas kernel here")


# speedup vs baseline: 1.2602x; 1.2602x over previous
import functools

import jax
import jax.numpy as jnp
import numpy as np
from jax.experimental import pallas as pl
from jax.experimental.pallas import tpu as pltpu

Q_N = 10000
KV_N = 10000
N_EDGES = 160000
NUM_CH = 256
NUM_HEADS = 8
HEAD_DIM = NUM_CH // NUM_HEADS
SIM_SCALE = 1.0 / np.sqrt(NUM_CH)

_ROWS_BLK = 1000


def _proj_body(q_ref, k_ref, g_ref, b_ref, wqt_ref, bq_ref, wkt_ref, bk_ref,
               qp_ref, kp_ref):
    x = q_ref[...]
    mu = jnp.mean(x, axis=1, keepdims=True)
    var = jnp.mean((x - mu) ** 2, axis=1, keepdims=True)
    xn = (x - mu) * jax.lax.rsqrt(var + 1e-5) * g_ref[...] + b_ref[...]
    qp_ref[...] = (
        jnp.dot(xn, wqt_ref[...], preferred_element_type=jnp.float32) + bq_ref[...]
    )
    kp_ref[...] = (
        jnp.dot(k_ref[...], wkt_ref[...], preferred_element_type=jnp.float32)
        + bk_ref[...]
    )


def _project(q, k, ln_gamma, ln_beta, Wq, bq, Wk, bk):
    grid = (Q_N // _ROWS_BLK,)
    row_spec = pl.BlockSpec((_ROWS_BLK, NUM_CH), lambda i: (i, 0))
    full_spec = pl.BlockSpec((NUM_CH, NUM_CH), lambda i: (0, 0))
    vec_spec = pl.BlockSpec((1, NUM_CH), lambda i: (0, 0))
    return pl.pallas_call(
        _proj_body,
        grid=grid,
        in_specs=[row_spec, row_spec, vec_spec, vec_spec, full_spec, vec_spec,
                  full_spec, vec_spec],
        out_specs=[row_spec, row_spec],
        out_shape=[
            jax.ShapeDtypeStruct((Q_N, NUM_CH), jnp.float32),
            jax.ShapeDtypeStruct((KV_N, NUM_CH), jnp.float32),
        ],
    )(q, k, ln_gamma.reshape(1, NUM_CH), ln_beta.reshape(1, NUM_CH),
      Wq.T, bq.reshape(1, NUM_CH), Wk.T, bk.reshape(1, NUM_CH))


def _final_body(acc_ref, res_ref, wot_ref, bo_ref, out_ref):
    out_ref[...] = (
        jnp.dot(acc_ref[...], wot_ref[...], preferred_element_type=jnp.float32)
        + bo_ref[...]
        + res_ref[...]
    )


def _final(acc, residual, Wo, bo):
    grid = (Q_N // _ROWS_BLK,)
    row_spec = pl.BlockSpec((_ROWS_BLK, NUM_CH), lambda i: (i, 0))
    full_spec = pl.BlockSpec((NUM_CH, NUM_CH), lambda i: (0, 0))
    vec_spec = pl.BlockSpec((1, NUM_CH), lambda i: (0, 0))
    return pl.pallas_call(
        _final_body,
        grid=grid,
        in_specs=[row_spec, row_spec, full_spec, vec_spec],
        out_specs=row_spec,
        out_shape=jax.ShapeDtypeStruct((Q_N, NUM_CH), jnp.float32),
    )(acc, residual, Wo.T, bo.reshape(1, NUM_CH))


def _edges_xla(qp, kp, src, dst):
    n = src.shape[0]
    k_e = jnp.take(kp, src, axis=0).reshape(n, NUM_HEADS, HEAD_DIM)
    q_e = jnp.take(qp, dst, axis=0).reshape(n, NUM_HEADS, HEAD_DIM)
    sim = (q_e * k_e).sum(axis=-1) * SIM_SCALE
    e = jnp.exp(sim)
    ws = jnp.full((Q_N, NUM_HEADS), 1e-8, dtype=jnp.float32).at[dst].add(e)
    weights = e / ws[dst]
    v_e = (k_e * weights[..., None]).reshape(n, NUM_CH)
    out = jnp.zeros((Q_N, NUM_CH), dtype=jnp.float32).at[dst].add(v_e)
    return out, weights


def kernel(q, k, v, associations, ln_gamma, ln_beta, Wq, bq, Wk, bk, Wo, bo):
    q = q.astype(jnp.float32)
    k = k.astype(jnp.float32)
    qp, kp = _project(q, k, ln_gamma, ln_beta, Wq, bq, Wk, bk)
    src = associations[:, 0]
    dst = associations[:, 1]
    acc, weights = _edges_xla(qp, kp, src, dst)
    out = _final(acc, q, Wo, bo)
    return out, weights[..., None]


# speedup vs baseline: 1.3016x; 1.0328x over previous
import functools

import jax
import jax.numpy as jnp
import numpy as np
from jax import lax
from jax.experimental import pallas as pl
from jax.experimental.pallas import tpu as pltpu
from jax.experimental.pallas import tpu_sc as plsc

Q_N = 10000
KV_N = 10000
N_EDGES = 160000
NUM_CH = 256
NUM_HEADS = 8
HEAD_DIM = NUM_CH // NUM_HEADS
SIM_SCALE = 1.0 / np.sqrt(NUM_CH)

_ROWS_BLK = 1000


def _proj_body(q_ref, k_ref, g_ref, b_ref, wqt_ref, bq_ref, wkt_ref, bk_ref,
               qp_ref, kp_ref):
    x = q_ref[...]
    mu = jnp.mean(x, axis=1, keepdims=True)
    var = jnp.mean((x - mu) ** 2, axis=1, keepdims=True)
    xn = (x - mu) * jax.lax.rsqrt(var + 1e-5) * g_ref[...] + b_ref[...]
    qp_ref[...] = (
        jnp.dot(xn, wqt_ref[...], preferred_element_type=jnp.float32) + bq_ref[...]
    )
    kp_ref[...] = (
        jnp.dot(k_ref[...], wkt_ref[...], preferred_element_type=jnp.float32)
        + bk_ref[...]
    )


def _project(q, k, ln_gamma, ln_beta, Wq, bq, Wk, bk):
    grid = (Q_N // _ROWS_BLK,)
    row_spec = pl.BlockSpec((_ROWS_BLK, NUM_CH), lambda i: (i, 0))
    full_spec = pl.BlockSpec((NUM_CH, NUM_CH), lambda i: (0, 0))
    vec_spec = pl.BlockSpec((1, NUM_CH), lambda i: (0, 0))
    return pl.pallas_call(
        _proj_body,
        grid=grid,
        in_specs=[row_spec, row_spec, vec_spec, vec_spec, full_spec, vec_spec,
                  full_spec, vec_spec],
        out_specs=[row_spec, row_spec],
        out_shape=[
            jax.ShapeDtypeStruct((Q_N, NUM_CH), jnp.float32),
            jax.ShapeDtypeStruct((KV_N, NUM_CH), jnp.float32),
        ],
    )(q, k, ln_gamma.reshape(1, NUM_CH), ln_beta.reshape(1, NUM_CH),
      Wq.T, bq.reshape(1, NUM_CH), Wk.T, bk.reshape(1, NUM_CH))


def _final_body(acc_ref, res_ref, wot_ref, bo_ref, out_ref):
    out_ref[...] = (
        jnp.dot(acc_ref[...], wot_ref[...], preferred_element_type=jnp.float32)
        + bo_ref[...]
        + res_ref[...]
    )


def _final(acc, residual, Wo, bo):
    grid = (Q_N // _ROWS_BLK,)
    row_spec = pl.BlockSpec((_ROWS_BLK, NUM_CH), lambda i: (i, 0))
    full_spec = pl.BlockSpec((NUM_CH, NUM_CH), lambda i: (0, 0))
    vec_spec = pl.BlockSpec((1, NUM_CH), lambda i: (0, 0))
    return pl.pallas_call(
        _final_body,
        grid=grid,
        in_specs=[row_spec, row_spec, full_spec, vec_spec],
        out_specs=row_spec,
        out_shape=jax.ShapeDtypeStruct((Q_N, NUM_CH), jnp.float32),
    )(acc, residual, Wo.T, bo.reshape(1, NUM_CH))


_NC = 2
_NS = 16
_NW = _NC * _NS
_CHUNK = 128
_NCHUNKS = N_EDGES // _CHUNK
_CPW = -(-_NCHUNKS // _NW)
_LANE = None


def _lanes():
    return lax.broadcasted_iota(jnp.int32, (16,), 0)


def _sc_pass1(qp, kp, src3, dst3):
    mesh = plsc.VectorSubcoreMesh(core_axis_name="c", subcore_axis_name="s")

    @functools.partial(
        pl.kernel,
        out_type=[
            jax.ShapeDtypeStruct((_NCHUNKS, _CHUNK, NUM_HEADS), jnp.float32),
        ],
        mesh=mesh,
        scratch_types=[
            pltpu.VMEM((_CHUNK,), jnp.int32),
            pltpu.VMEM((_CHUNK,), jnp.int32),
            pltpu.VMEM((_CHUNK, NUM_CH), jnp.float32),
            pltpu.VMEM((_CHUNK, NUM_CH), jnp.float32),
            pltpu.VMEM((_CHUNK, NUM_HEADS), jnp.float32),
            pltpu.VMEM((16, 16), jnp.float32),
            pltpu.SemaphoreType.DMA,
            pltpu.SemaphoreType.DMA,
        ],
        compiler_params=pltpu.CompilerParams(needs_layout_passes=False),
    )
    def k1(qp_hbm, kp_hbm, src_hbm, dst_hbm, e_hbm,
           src_v, dst_v, qrows, krows, ebuf, sbuf, sem1, sem2):
        c = lax.axis_index("c")
        s = lax.axis_index("s")
        wid = s * _NC + c
        lane = _lanes()

        def chunk_body(j, _):
            cid = j * _NW + wid

            @pl.when(cid < _NCHUNKS)
            def _():
                _do_chunk(cid)
            return _

        def _do_chunk(cid):
            pltpu.sync_copy(src_hbm.at[cid], src_v)
            pltpu.sync_copy(dst_hbm.at[cid], dst_v)
            d1 = pltpu.async_copy(kp_hbm.at[src_v], krows, sem1)
            d2 = pltpu.async_copy(qp_hbm.at[dst_v], qrows, sem2)
            d1.wait()
            d2.wait()

            def pair_body(i, _):
                r0 = i * 2
                r1 = r0 + 1
                for h in range(NUM_HEADS):
                    o = h * HEAD_DIM
                    pa = (qrows[r0, pl.ds(o, 16)] * krows[r0, pl.ds(o, 16)]
                          + qrows[r0, pl.ds(o + 16, 16)] * krows[r0, pl.ds(o + 16, 16)])
                    pb = (qrows[r1, pl.ds(o, 16)] * krows[r1, pl.ds(o, 16)]
                          + qrows[r1, pl.ds(o + 16, 16)] * krows[r1, pl.ds(o + 16, 16)])
                    sbuf[h] = pa
                    sbuf[h + 8] = pb
                sv = jnp.zeros((16,), jnp.float32)
                for ccol in range(16):
                    sv = sv + plsc.load_gather(
                        sbuf, [lane, jnp.full((16,), ccol, jnp.int32)])
                ev = jnp.exp(sv * SIM_SCALE)
                rowi = jnp.where(lane < 8, r0, r1)
                coli = lax.bitwise_and(lane, 7)
                plsc.store_scatter(ebuf, [rowi, coli], ev)
                return _

            lax.fori_loop(0, _CHUNK // 2, pair_body, None)
            pltpu.sync_copy(ebuf, e_hbm.at[cid])

        lax.fori_loop(0, _CPW, chunk_body, None)

    return k1(qp, kp, src3, dst3)


def _edges_xla(qp, kp, src, dst):
    n = src.shape[0]
    k_e = jnp.take(kp, src, axis=0).reshape(n, NUM_HEADS, HEAD_DIM)
    q_e = jnp.take(qp, dst, axis=0).reshape(n, NUM_HEADS, HEAD_DIM)
    sim = (q_e * k_e).sum(axis=-1) * SIM_SCALE
    e = jnp.exp(sim)
    ws = jnp.full((Q_N, NUM_HEADS), 1e-8, dtype=jnp.float32).at[dst].add(e)
    weights = e / ws[dst]
    v_e = (k_e * weights[..., None]).reshape(n, NUM_CH)
    out = jnp.zeros((Q_N, NUM_CH), dtype=jnp.float32).at[dst].add(v_e)
    return out, weights


def kernel(q, k, v, associations, ln_gamma, ln_beta, Wq, bq, Wk, bk, Wo, bo):
    q = q.astype(jnp.float32)
    k = k.astype(jnp.float32)
    qp, kp = _project(q, k, ln_gamma, ln_beta, Wq, bq, Wk, bk)
    src = associations[:, 0]
    dst = associations[:, 1]
    src3 = src.reshape(_NCHUNKS, _CHUNK)
    dst3 = dst.reshape(_NCHUNKS, _CHUNK)
    (E3,) = _sc_pass1(qp, kp, src3, dst3)
    E = E3.reshape(N_EDGES, NUM_HEADS)
    ws = jnp.full((Q_N, NUM_HEADS), 1e-8, jnp.float32).at[dst].add(E)
    weights = E / ws[dst]
    k_e = jnp.take(kp, src, axis=0).reshape(N_EDGES, NUM_HEADS, HEAD_DIM)
    v_e = (k_e * weights[..., None]).reshape(N_EDGES, NUM_CH)
    acc = jnp.zeros((Q_N, NUM_CH), dtype=jnp.float32).at[dst].add(v_e)
    out = _final(acc, q, Wo, bo)
    return out, weights[..., None]
